# Initial kernel scaffold; baseline (speedup 1.0000x reference)
import jax
import jax.numpy as jnp
from jax.experimental import pallas as pl


def kernel(x):
    raise NotImplementedError("write your pallas kernel here")


# speedup vs baseline: 5.7928x; 5.7928x over previous
import jax
import jax.numpy as jnp
from jax import lax
from jax.experimental import pallas as pl
from jax.experimental.pallas import tpu as pltpu
from jax.experimental.pallas import tpu_sc as plsc

ROWS = 128
COLS = 32768
K = 64
LANES = 16
NV = COLS // LANES
UNROLL = 8
NWORKERS = 32
ROWS_PER = ROWS // NWORKERS
CAP = 2048
NCV = CAP // LANES

def _keys(xb):
    sign = jnp.uint32(0x80000000)
    b = lax.bitcast_convert_type(xb, jnp.uint32)
    neg = b >= sign
    return jnp.where(neg, ~b, b | sign)


def _body(x_hbm, out_hbm, xv, kv, cv, ov):
    wid = lax.axis_index("s") * 2 + lax.axis_index("c")

    for r in range(ROWS_PER):
        row = wid * ROWS_PER + r
        pltpu.sync_copy(x_hbm.at[row], xv)

        def a_body(i, carry):
            a0, a1, a2, a3 = carry
            for u in range(UNROLL):
                sl = pl.ds((i * UNROLL + u) * LANES, LANES)
                t = _keys(xv[sl])
                kv[sl] = t
                m = jnp.maximum(a0, t); t = jnp.minimum(a0, t); a0 = m
                m = jnp.maximum(a1, t); t = jnp.minimum(a1, t); a1 = m
                m = jnp.maximum(a2, t); t = jnp.minimum(a2, t); a2 = m
                a3 = jnp.maximum(a3, t)
            return (a0, a1, a2, a3)

        zero4 = [jnp.zeros((LANES,), jnp.uint32) for _ in range(4)]
        a0, a1, a2, a3 = lax.fori_loop(0, NV // UNROLL, a_body, tuple(zero4))
        t2 = jnp.min(a3)
        t2s = jnp.full((LANES,), t2, dtype=jnp.uint32)

        def fill_body(i, _):
            for u in range(UNROLL):
                cv[pl.ds((i * UNROLL + u) * LANES, LANES)] = (
                    jnp.zeros((LANES,), jnp.uint32))
            return 0

        lax.fori_loop(0, NCV // UNROLL, fill_body, 0)

        def b_body(i, off):
            for u in range(UNROLL):
                sl = pl.ds((i * UNROLL + u) * LANES, LANES)
                kx = kv[sl]
                msk = kx >= t2s
                plsc.store_compressed(cv.at[pl.ds(off, LANES)], kx, mask=msk)
                cnt = plsc.all_reduce_population_count(msk)
                off = jnp.minimum(off + cnt[0], jnp.int32(CAP - LANES))
            return off

        lax.fori_loop(0, NV // UNROLL, b_body, jnp.int32(0))

        def count_ge(ms):
            def cb(i, acc):
                for u in range(4):
                    sl = pl.ds((i * 4 + u) * LANES, LANES)
                    acc = acc + (cv[sl] >= ms).astype(jnp.int32)
                return acc
            acc = lax.fori_loop(0, NCV // 4, cb,
                                jnp.zeros((LANES,), jnp.int32))
            return jnp.sum(acc)

        def s_body(j, carry):
            lo, hi = carry
            mid = lo + (hi - lo) // jnp.uint32(2)
            cnt = count_ge(jnp.full((LANES,), mid, dtype=jnp.uint32))
            ge = cnt >= K
            return (jnp.where(ge, mid, lo), jnp.where(ge, hi, mid))

        lo, _ = lax.fori_loop(0, 32, s_body, (t2, jnp.uint32(0xFFFFFFFF)))
        thr = jnp.full((LANES,), lo, dtype=jnp.uint32)

        def mask_body(i, _):
            for u in range(UNROLL):
                sl = pl.ds((i * UNROLL + u) * LANES, LANES)
                ov[sl] = jnp.where(kv[sl] >= thr, xv[sl], jnp.float32(0.0))
            return 0

        lax.fori_loop(0, NV // UNROLL, mask_body, 0)
        pltpu.sync_copy(ov, out_hbm.at[row])


def kernel(x):
    mesh = plsc.VectorSubcoreMesh(core_axis_name="c", subcore_axis_name="s")
    f = pl.kernel(
        _body,
        mesh=mesh,
        out_type=jax.ShapeDtypeStruct((ROWS, COLS), jnp.float32),
        scratch_types=[
            pltpu.VMEM((COLS,), jnp.float32),
            pltpu.VMEM((COLS,), jnp.uint32),
            pltpu.VMEM((CAP,), jnp.uint32),
            pltpu.VMEM((COLS,), jnp.float32),
        ],
        compiler_params=pltpu.CompilerParams(needs_layout_passes=False),
    )
    return f(x)


# speedup vs baseline: 7.7341x; 1.3351x over previous
import jax
import jax.numpy as jnp
from jax import lax
from jax.experimental import pallas as pl
from jax.experimental.pallas import tpu as pltpu
from jax.experimental.pallas import tpu_sc as plsc

ROWS = 128
COLS = 32768
K = 64
LANES = 16
NV = COLS // LANES
UNROLL = 8
NWORKERS = 32
ROWS_PER = ROWS // NWORKERS
CAP = 1024
NCV = CAP // LANES


def _keys(xb):
    sign = jnp.uint32(0x80000000)
    b = lax.bitcast_convert_type(xb, jnp.uint32)
    neg = b >= sign
    return jnp.where(neg, ~b, b | sign)


def _process_row(xb, cv, ov, wait_out=None):
    def a_body(i, carry):
        a0, a1, a2, a3 = carry
        for g in range(UNROLL // 4):
            base = (i * UNROLL + g * 4) * LANES
            v0 = xb[pl.ds(base, LANES)]
            v1 = xb[pl.ds(base + LANES, LANES)]
            v2 = xb[pl.ds(base + 2 * LANES, LANES)]
            v3 = xb[pl.ds(base + 3 * LANES, LANES)]
            t = jnp.maximum(jnp.maximum(v0, v1), jnp.maximum(v2, v3))
            m = jnp.maximum(a0, t); t = jnp.minimum(a0, t); a0 = m
            m = jnp.maximum(a1, t); t = jnp.minimum(a1, t); a1 = m
            m = jnp.maximum(a2, t); t = jnp.minimum(a2, t); a2 = m
            a3 = jnp.maximum(a3, t)
        return (a0, a1, a2, a3)

    ninf = [jnp.full((LANES,), -jnp.inf, jnp.float32) for _ in range(4)]
    _, _, _, a3 = lax.fori_loop(0, NV // UNROLL, a_body, tuple(ninf))
    t2s = jnp.full((LANES,), jnp.min(a3), dtype=jnp.float32)

    def fill_body(i, _):
        for u in range(UNROLL):
            cv[pl.ds((i * UNROLL + u) * LANES, LANES)] = (
                jnp.zeros((LANES,), jnp.uint32))
        return 0

    lax.fori_loop(0, NCV // UNROLL, fill_body, 0)

    def b_body(i, off):
        for u in range(UNROLL):
            sl = pl.ds((i * UNROLL + u) * LANES, LANES)
            x = xb[sl]
            msk = x >= t2s
            plsc.store_compressed(cv.at[pl.ds(off, LANES)], _keys(x),
                                  mask=msk)
            cnt = plsc.all_reduce_population_count(msk)
            off = jnp.minimum(off + cnt[0], jnp.int32(CAP - LANES))
        return off

    lax.fori_loop(0, NV // UNROLL, b_body, jnp.int32(0))

    def count_ge(ms):
        def cb(i, acc):
            for u in range(4):
                sl = pl.ds((i * 4 + u) * LANES, LANES)
                acc = acc + (cv[sl] >= ms).astype(jnp.int32)
            return acc
        acc = lax.fori_loop(0, NCV // 4, cb, jnp.zeros((LANES,), jnp.int32))
        return jnp.sum(acc)

    def s_body(j, carry):
        lo, hi = carry
        mid = lo + (hi - lo) // jnp.uint32(2)
        cnt = count_ge(jnp.full((LANES,), mid, dtype=jnp.uint32))
        ge = cnt >= K
        return (jnp.where(ge, mid, lo), jnp.where(ge, hi, mid))

    lo0 = _keys(t2s)[0]
    lo, _ = lax.fori_loop(0, 32, s_body, (lo0, jnp.uint32(0xFFFFFFFF)))

    thr = jnp.full((LANES,), lo, dtype=jnp.uint32)
    sign = jnp.uint32(0x80000000)
    thr_bits = jnp.where(thr >= sign, thr ^ sign, ~thr)
    thrf = plsc.bitcast(thr_bits, jnp.float32)

    if wait_out is not None:
        wait_out()

    def mask_body(i, _):
        for u in range(UNROLL):
            sl = pl.ds((i * UNROLL + u) * LANES, LANES)
            x = xb[sl]
            ov[sl] = jnp.where(x >= thrf, x, jnp.float32(0.0))
        return 0

    lax.fori_loop(0, NV // UNROLL, mask_body, 0)


def _body(x_hbm, out_hbm, x0, x1, ov, cv, sin0, sin1, sout):
    wid = lax.axis_index("s") * 2 + lax.axis_index("c")
    row0 = wid * ROWS_PER
    bufs = (x0, x1)
    sems = (sin0, sin1)

    in_handles = [None, None]
    in_handles[0] = pltpu.async_copy(x_hbm.at[row0], x0, sin0)
    out_handle = None
    for r in range(ROWS_PER):
        xb = bufs[r % 2]
        if r + 1 < ROWS_PER:
            in_handles[(r + 1) % 2] = pltpu.async_copy(
                x_hbm.at[row0 + r + 1], bufs[(r + 1) % 2],
                sems[(r + 1) % 2])
        in_handles[r % 2].wait()
        _process_row(xb, cv, ov,
                     wait_out=out_handle.wait if out_handle else None)
        out_handle = pltpu.async_copy(ov, out_hbm.at[row0 + r], sout)
    out_handle.wait()


def kernel(x):
    mesh = plsc.VectorSubcoreMesh(core_axis_name="c", subcore_axis_name="s")
    f = pl.kernel(
        _body,
        mesh=mesh,
        out_type=jax.ShapeDtypeStruct((ROWS, COLS), jnp.float32),
        scratch_types=[
            pltpu.VMEM((COLS,), jnp.float32),
            pltpu.VMEM((COLS,), jnp.float32),
            pltpu.VMEM((COLS,), jnp.float32),
            pltpu.VMEM((CAP,), jnp.uint32),
            pltpu.SemaphoreType.DMA,
            pltpu.SemaphoreType.DMA,
            pltpu.SemaphoreType.DMA,
        ],
        compiler_params=pltpu.CompilerParams(needs_layout_passes=False),
    )
    return f(x)
